# baseline (speedup 1.0000x reference)
import functools

import jax
import jax.numpy as jnp
from jax import lax
from jax.experimental import pallas as pl
from jax.experimental.pallas import tpu as pltpu
from jax.experimental.pallas import tpu_sc as plsc

N_NODES = 10000
N_EDGES = 320000
EDIM = 16
NDIM_IN = 128
NDIM_OUT = 128
ACCW = 32

NWORK = 32
EPT = N_EDGES // NWORK
SUB = 100
CH = 1000
NCHH = 5
RPC = CH // SUB
IPT = EPT // SUB
HROWS_PER_TILE = EPT // 2 // 4
CH_ROWS = CH // 4
N_PAD = 10240
ROWS_PER_TILE = N_PAD // 16

MSG_BLK = 4000


def _msg_body(ef_ref, nwx_ref, pe_ref, po_ref, oh_ref, ev_ref, od_ref):
    scaled = ef_ref[...] * nwx_ref[...]
    oh = oh_ref[...]
    ev_ref[...] = jnp.dot(scaled, pe_ref[...],
                          preferred_element_type=jnp.float32) + oh
    od_ref[...] = jnp.dot(scaled, po_ref[...],
                          preferred_element_type=jnp.float32) + oh


def _make_msg(ef_r, nwx_r, pe, po, oh):
    grid = (N_EDGES // 8) // MSG_BLK
    return pl.pallas_call(
        _msg_body,
        grid=(grid,),
        in_specs=[
            pl.BlockSpec((MSG_BLK, 128), lambda i: (i, 0)),
            pl.BlockSpec((MSG_BLK, 128), lambda i: (i, 0)),
            pl.BlockSpec((128, 128), lambda i: (0, 0)),
            pl.BlockSpec((128, 128), lambda i: (0, 0)),
            pl.BlockSpec((1, 128), lambda i: (0, 0)),
        ],
        out_specs=[
            pl.BlockSpec((MSG_BLK, 128), lambda i: (i, 0)),
            pl.BlockSpec((MSG_BLK, 128), lambda i: (i, 0)),
        ],
        out_shape=[jax.ShapeDtypeStruct((N_EDGES // 8, 128), jnp.float32),
                   jax.ShapeDtypeStruct((N_EDGES // 8, 128), jnp.float32)],
    )(ef_r, nwx_r, pe, po, oh)


def _sc_scatter_body(mev_hbm, mod_hbm, dst_hbm, zeros_hbm, out_hbm,
                     msg_a, msg_b, dst_v, acc_sh, sem_in, sem_sc):
    cid = lax.axis_index("c")
    sid = lax.axis_index("s")
    wid = cid * 16 + sid

    zbase = sid * ROWS_PER_TILE
    pltpu.sync_copy(zeros_hbm.at[pl.ds(zbase, ROWS_PER_TILE)],
                    acc_sh.at[pl.ds(zbase, ROWS_PER_TILE)])
    pltpu.sync_copy(dst_hbm.at[wid], dst_v)
    plsc.subcore_barrier()

    bufs = [msg_a, msg_b]
    pending = [[], []]
    stage = [None, None]

    def start_stage(ci):
        src = mev_hbm if ci < NCHH else mod_hbm
        e0 = wid * (EPT // 2) + (ci % NCHH) * CH
        return pltpu.async_copy(src.at[pl.ds(e0, CH)],
                                bufs[ci % 2], sem_in)

    stage[0] = start_stage(0)
    for ci in range(2 * NCHH):
        b = ci % 2
        nb = (ci + 1) % 2
        if ci + 1 < 2 * NCHH:
            for h in pending[nb]:
                h.wait()
            pending[nb] = []
            stage[nb] = start_stage(ci + 1)
        stage[b].wait()
        hs = []
        for j in range(RPC):
            hs.append(pltpu.async_copy(bufs[b].at[pl.ds(j * SUB, SUB)],
                                       acc_sh.at[dst_v.at[ci * RPC + j]],
                                       sem_sc, add=True))
        pending[b] = hs

    for bb in range(2):
        for h in pending[bb]:
            h.wait()

    plsc.subcore_barrier()
    pltpu.sync_copy(acc_sh.at[pl.ds(zbase, ROWS_PER_TILE)],
                    out_hbm.at[cid, pl.ds(zbase, ROWS_PER_TILE)])


_sc_scatter = functools.partial(
    pl.kernel,
    out_type=jax.ShapeDtypeStruct((2, N_PAD, ACCW), jnp.float32),
    mesh=plsc.VectorSubcoreMesh(core_axis_name="c", subcore_axis_name="s"),
    compiler_params=pltpu.CompilerParams(use_tc_tiling_on_sc=False,
                                         needs_layout_passes=False),
    scratch_types=[
        pltpu.VMEM((CH, ACCW), jnp.float32),
        pltpu.VMEM((CH, ACCW), jnp.float32),
        pltpu.VMEM((IPT, SUB), jnp.int32),
        pltpu.VMEM_SHARED((N_PAD, ACCW), jnp.float32),
        pltpu.SemaphoreType.DMA,
        pltpu.SemaphoreType.DMA,
    ],
)(_sc_scatter_body)


def _final_body(parts_ref, nf_ref, wt_ref, b_ref, out_ref):
    s = parts_ref[0] + parts_ref[1]
    deg = jnp.maximum(s[:N_NODES, EDIM:EDIM + 1], 1.0)
    h_neigh = s[:N_NODES, :EDIM] / deg
    h = jnp.concatenate([nf_ref[...], h_neigh], axis=1)
    acc = jnp.dot(h, wt_ref[...], preferred_element_type=jnp.float32)
    out_ref[...] = jnp.maximum(acc + b_ref[...], 0.0)


def _final(parts, nf2, wt, b2):
    return pl.pallas_call(
        _final_body,
        out_shape=jax.ShapeDtypeStruct((N_NODES, NDIM_OUT), jnp.float32),
    )(parts, nf2, wt, b2)


def kernel(nfeats, efeats, edge_index, norm_weight, W, b):
    ef_r = efeats.reshape(N_EDGES // 8, 128)
    nwx_r = jnp.broadcast_to(norm_weight[:, None],
                             (N_EDGES, EDIM)).reshape(N_EDGES // 8, 128)

    c = jnp.arange(128)
    sel = (c % ACCW) < EDIM
    ke = jnp.where(sel, EDIM * (c // ACCW) + (c % EDIM), -1)
    pe = (jnp.arange(128)[:, None] == ke[None, :]).astype(jnp.float32)
    po = (jnp.arange(128)[:, None] == jnp.where(sel, 64 + ke, -2)[None, :]
          ).astype(jnp.float32)
    oh = ((c % ACCW) == EDIM).astype(jnp.float32).reshape(1, 128)

    dsti = edge_index[1].astype(jnp.int32).reshape(NWORK, EPT // 8, 8)
    de = dsti[:, :, :4].reshape(NWORK, IPT // 2, SUB)
    do = dsti[:, :, 4:].reshape(NWORK, IPT // 2, SUB)
    dst = jnp.concatenate([de, do], axis=1)

    zeros = jnp.zeros((N_PAD, ACCW), jnp.float32)
    wt = W.T
    b2 = b.reshape(1, NDIM_OUT)

    mev, mod = _make_msg(ef_r, nwx_r, pe, po, oh)
    mev = mev.reshape(N_EDGES // 2, ACCW)
    mod = mod.reshape(N_EDGES // 2, ACCW)
    parts = _sc_scatter(mev, mod, dst, zeros)
    out2 = _final(parts, nfeats.reshape(N_NODES, NDIM_IN), wt, b2)
    return out2.reshape(N_NODES, 1, NDIM_OUT)


# speedup vs baseline: 1.7198x; 1.7198x over previous
import functools

import jax
import jax.numpy as jnp
from jax import lax
from jax.experimental import pallas as pl
from jax.experimental.pallas import tpu as pltpu
from jax.experimental.pallas import tpu_sc as plsc

N_NODES = 10000
N_EDGES = 320000
EDIM = 16
NDIM_IN = 128
NDIM_OUT = 128
ACCW = 32

NWORK = 32
EPT = N_EDGES // NWORK
SUB = 100
CH = 1000
NCH = EPT // CH
RPC = CH // SUB
IPT = EPT // SUB
N_PAD = 10240
ROWS_PER_TILE = N_PAD // 16


def _sc_scatter_body(ef_hbm, nw_hbm, dst_hbm, zeros_hbm, out_hbm,
                     ef_a, ef_b, msg_a, msg_b, nw_a, nw_b, dst_v, acc_sh,
                     sem_in, sem_sc):
    cid = lax.axis_index("c")
    sid = lax.axis_index("s")
    wid = cid * 16 + sid

    zbase = sid * ROWS_PER_TILE
    pltpu.sync_copy(zeros_hbm.at[pl.ds(zbase, ROWS_PER_TILE)],
                    acc_sh.at[pl.ds(zbase, ROWS_PER_TILE)])
    pltpu.sync_copy(dst_hbm.at[wid], dst_v)

    onehot = jnp.where(lax.iota(jnp.int32, 16) == 0, 1.0, 0.0)

    @plsc.parallel_loop(0, CH, unroll=8)
    def _(e):
        msg_a[e, pl.ds(EDIM, EDIM)] = onehot
        msg_b[e, pl.ds(EDIM, EDIM)] = onehot

    plsc.subcore_barrier()

    efs = [ef_a, ef_b]
    msgs = [msg_a, msg_b]
    nws = [nw_a, nw_b]
    pending = [[], []]
    stage = [None, None]

    def start_stage(ci):
        base = wid * EPT + ci * CH
        return (pltpu.async_copy(ef_hbm.at[pl.ds(base, CH)],
                                 efs[ci % 2], sem_in),
                pltpu.async_copy(nw_hbm.at[pl.ds(base, CH)],
                                 nws[ci % 2], sem_in))

    stage[0] = start_stage(0)
    for ci in range(NCH):
        b = ci % 2
        nb = (ci + 1) % 2
        if ci + 1 < NCH:
            for h in pending[nb]:
                h.wait()
            pending[nb] = []
            stage[nb] = start_stage(ci + 1)
        for h in stage[b]:
            h.wait()
        efb = efs[b]
        msgb = msgs[b]
        nwb = nws[b]

        @plsc.parallel_loop(0, CH, unroll=8)
        def _(e):
            nwv = plsc.load_gather(nwb, [jnp.full((16,), e, dtype=jnp.int32)])
            msgb[e, pl.ds(0, EDIM)] = efb[e, :] * nwv

        hs = []
        for j in range(RPC):
            hs.append(pltpu.async_copy(msgb.at[pl.ds(j * SUB, SUB)],
                                       acc_sh.at[dst_v.at[ci * RPC + j]],
                                       sem_sc, add=True))
        pending[b] = hs

    for bb in range(2):
        for h in pending[bb]:
            h.wait()

    plsc.subcore_barrier()
    pltpu.sync_copy(acc_sh.at[pl.ds(zbase, ROWS_PER_TILE)],
                    out_hbm.at[cid, pl.ds(zbase, ROWS_PER_TILE)])


_sc_scatter = functools.partial(
    pl.kernel,
    out_type=jax.ShapeDtypeStruct((2, N_PAD, ACCW), jnp.float32),
    mesh=plsc.VectorSubcoreMesh(core_axis_name="c", subcore_axis_name="s"),
    compiler_params=pltpu.CompilerParams(use_tc_tiling_on_sc=False,
                                         needs_layout_passes=False),
    scratch_types=[
        pltpu.VMEM((CH, EDIM), jnp.float32),
        pltpu.VMEM((CH, EDIM), jnp.float32),
        pltpu.VMEM((CH, ACCW), jnp.float32),
        pltpu.VMEM((CH, ACCW), jnp.float32),
        pltpu.VMEM((CH,), jnp.float32),
        pltpu.VMEM((CH,), jnp.float32),
        pltpu.VMEM((IPT, SUB), jnp.int32),
        pltpu.VMEM_SHARED((N_PAD, ACCW), jnp.float32),
        pltpu.SemaphoreType.DMA,
        pltpu.SemaphoreType.DMA,
    ],
)(_sc_scatter_body)


def _final_body(parts_ref, nf_ref, wt_ref, b_ref, out_ref):
    s = parts_ref[0] + parts_ref[1]
    deg = jnp.maximum(s[:N_NODES, EDIM:EDIM + 1], 1.0)
    h_neigh = s[:N_NODES, :EDIM] / deg
    h = jnp.concatenate([nf_ref[...], h_neigh], axis=1)
    acc = jnp.dot(h, wt_ref[...], preferred_element_type=jnp.float32)
    out_ref[...] = jnp.maximum(acc + b_ref[...], 0.0)


def _final(parts, nf2, wt, b2):
    return pl.pallas_call(
        _final_body,
        out_shape=jax.ShapeDtypeStruct((N_NODES, NDIM_OUT), jnp.float32),
    )(parts, nf2, wt, b2)


def kernel(nfeats, efeats, edge_index, norm_weight, W, b):
    ef2 = efeats.reshape(N_EDGES, EDIM)
    dst = edge_index[1].astype(jnp.int32).reshape(NWORK, IPT, SUB)
    zeros = jnp.zeros((N_PAD, ACCW), jnp.float32)
    wt = W.T
    b2 = b.reshape(1, NDIM_OUT)

    parts = _sc_scatter(ef2, norm_weight, dst, zeros)
    out2 = _final(parts, nfeats.reshape(N_NODES, NDIM_IN), wt, b2)
    return out2.reshape(N_NODES, 1, NDIM_OUT)
